# baseline (speedup 1.0000x reference)
import jax
import jax.numpy as jnp
from jax.experimental import pallas as pl

_NUM_ITEMS = 100000
_ROW_BLOCK = 64


def _onehot_body(it_ref, o_ref):
    cols = jax.lax.broadcasted_iota(jnp.int32, o_ref.shape, 1)
    o_ref[...] = (cols == it_ref[...]).astype(jnp.float32)


def kernel(x):
    B = x.shape[0]
    it = jax.random.randint(jax.random.key(42), (B,), 1, _NUM_ITEMS)
    it2 = it.astype(jnp.int32).reshape(B, 1)
    grid = (B // _ROW_BLOCK,)
    out = pl.pallas_call(
        _onehot_body,
        grid=grid,
        in_specs=[pl.BlockSpec((_ROW_BLOCK, 1), lambda i: (i, 0))],
        out_specs=pl.BlockSpec((_ROW_BLOCK, _NUM_ITEMS), lambda i: (i, 0)),
        out_shape=jax.ShapeDtypeStruct((B, _NUM_ITEMS), jnp.float32),
    )(it2)
    return out


# speedup vs baseline: 1.0012x; 1.0012x over previous
import jax
import jax.numpy as jnp
from jax.experimental import pallas as pl

_NUM_ITEMS = 100000
_ROW_BLOCK = 16


def _onehot_body(it_ref, o_ref):
    cols = jax.lax.broadcasted_iota(jnp.int32, o_ref.shape, 1)
    o_ref[...] = (cols == it_ref[...]).astype(jnp.float32)


def kernel(x):
    B = x.shape[0]
    it = jax.random.randint(jax.random.key(42), (B,), 1, _NUM_ITEMS)
    it2 = it.astype(jnp.int32).reshape(B, 1)
    grid = (B // _ROW_BLOCK,)
    out = pl.pallas_call(
        _onehot_body,
        grid=grid,
        in_specs=[pl.BlockSpec((_ROW_BLOCK, 1), lambda i: (i, 0))],
        out_specs=pl.BlockSpec((_ROW_BLOCK, _NUM_ITEMS), lambda i: (i, 0)),
        out_shape=jax.ShapeDtypeStruct((B, _NUM_ITEMS), jnp.float32),
    )(it2)
    return out
